# Initial kernel scaffold; baseline (speedup 1.0000x reference)
import jax
import jax.numpy as jnp
from jax.experimental import pallas as pl


def kernel(x, cw1, cb1, cw2, cb2, cw3, cb3, fw1, fb1, fw2, fb2):
    raise NotImplementedError("write your pallas kernel here")


# speedup vs baseline: 6.2548x; 6.2548x over previous
import functools

import jax
import jax.numpy as jnp
from jax.experimental import pallas as pl
from jax.experimental.pallas import tpu as pltpu

_BF = jnp.bfloat16
_F32 = jnp.float32


def _shift_lanes(v, d):
    if d == 0:
        return v
    if d < 0:
        return jnp.concatenate([v[..., 1:], v[..., :1]], axis=-1)
    return jnp.concatenate([v[..., -1:], v[..., :-1]], axis=-1)


def _build_shifted(x, cin, cpad, r_ref):
    hin2, _, lanes = x.shape
    for dx in range(3):
        s = _shift_lanes(x, 1 - dx)
        if cpad != cin:
            s = jnp.concatenate(
                [s, jnp.zeros((hin2, cpad - cin, lanes), x.dtype)], axis=1)
        r_ref[dx] = s


def _chunk_patch(r_ref, rows):
    cols = [
        jnp.concatenate(
            [r_ref[dx, y + dy] for dy in range(3) for dx in range(3)], axis=0)
        for y in rows
    ]
    return cols[0] if len(cols) == 1 else jnp.concatenate(cols, axis=1)


def _pooled_chunk(r_ref, wm, bias, sel, p0, pp, lanes):
    rows_e = [2 * (p0 + i) for i in range(pp)]
    ae = jnp.dot(wm, _chunk_patch(r_ref, rows_e), preferred_element_type=_F32)
    ao = jnp.dot(wm, _chunk_patch(r_ref, [r + 1 for r in rows_e]),
                 preferred_element_type=_F32)
    m = jnp.maximum(ae, ao)
    mm = jnp.maximum(m, _shift_lanes(m, -1))
    pooled = jnp.maximum(mm + bias, 0.0).astype(_BF)
    return [
        jnp.dot(pooled[:, i * lanes:(i + 1) * lanes], sel,
                preferred_element_type=_F32).astype(_BF)
        for i in range(pp)
    ]


def _conv_pool_kernel(cin, cpad, hout, pp, wvalid, x_ref, wm_ref, b_ref,
                      s_ref, o_ref, r_ref):
    lanes = x_ref.shape[3]
    _build_shifted(x_ref[0], cin, cpad, r_ref)
    wm = wm_ref[...]
    bias = b_ref[...]
    sel = s_ref[...]
    cout = o_ref.shape[2]
    o_ref[0, 0] = jnp.zeros((cout, 128), o_ref.dtype)
    o_ref[0, hout + 1] = jnp.zeros((cout, 128), o_ref.dtype)
    for p0 in range(0, hout, pp):
        rows = _pooled_chunk(r_ref, wm, bias, sel, p0, pp, lanes)
        for i in range(pp):
            o_ref[0, 1 + p0 + i] = rows[i]


def _conv_pool_nhwc_kernel(cin, cpad, hout, pp, wvalid, x_ref, wm_ref, b_ref,
                           s_ref, i_ref, o_ref, r_ref):
    lanes = x_ref.shape[3]
    _build_shifted(x_ref[0], cin, cpad, r_ref)
    wm = wm_ref[...]
    bias = b_ref[...]
    sel = s_ref[...]
    ident = i_ref[...]
    for p0 in range(0, hout, pp):
        rows = _pooled_chunk(r_ref, wm, bias, sel, p0, pp, lanes)
        d = jnp.concatenate(rows, axis=1)
        t = jax.lax.dot_general(d, ident, (((0,), (0,)), ((), ())),
                                preferred_element_type=_F32)
        tb = t.astype(o_ref.dtype)
        for i in range(pp):
            o_ref[0, p0 + i] = tb[i * 128:i * 128 + wvalid, :]


def _conv_stage(x4, wm, bias, sel, *, cin, cpad, hout, wvalid, pp,
                nhwc=False, ident=None):
    bsz, hin2, _, lanes = x4.shape
    cout = wm.shape[0]
    if nhwc:
        kern = functools.partial(_conv_pool_nhwc_kernel, cin, cpad, hout, pp,
                                 wvalid)
        out_shape = jax.ShapeDtypeStruct((bsz, hout, wvalid, cout), _BF)
        out_spec = pl.BlockSpec((1, hout, wvalid, cout),
                                lambda b: (b, 0, 0, 0))
        extra = [pl.BlockSpec((128, 128), lambda b: (0, 0))]
        ops = (x4, wm, bias, sel, ident)
    else:
        kern = functools.partial(_conv_pool_kernel, cin, cpad, hout, pp,
                                 wvalid)
        out_shape = jax.ShapeDtypeStruct((bsz, hout + 2, cout, 128), _BF)
        out_spec = pl.BlockSpec((1, hout + 2, cout, 128),
                                lambda b: (b, 0, 0, 0))
        extra = []
        ops = (x4, wm, bias, sel)
    return pl.pallas_call(
        kern,
        out_shape=out_shape,
        grid=(bsz,),
        in_specs=[
            pl.BlockSpec((1, hin2, cin, lanes), lambda b: (b, 0, 0, 0)),
            pl.BlockSpec(wm.shape, lambda b: (0, 0)),
            pl.BlockSpec((cout, 1), lambda b: (0, 0)),
            pl.BlockSpec(sel.shape, lambda b: (0, 0)),
        ] + extra,
        out_specs=out_spec,
        scratch_shapes=[pltpu.VMEM((3, hin2, cpad, lanes), _BF)],
        compiler_params=pltpu.CompilerParams(
            dimension_semantics=("parallel",),
            vmem_limit_bytes=64 * 1024 * 1024),
    )(*ops)


def _fc1_kernel(x_ref, w_ref, o_ref, acc_ref):
    k = pl.program_id(1)

    @pl.when(k == 0)
    def _():
        acc_ref[...] = jnp.zeros(acc_ref.shape, acc_ref.dtype)

    acc_ref[...] += jnp.dot(x_ref[...], w_ref[...],
                            preferred_element_type=_F32)

    @pl.when(k == pl.num_programs(1) - 1)
    def _():
        o_ref[0] = acc_ref[...]


def _fc2_kernel(p_ref, b1_ref, w2_ref, b2_ref, o_ref):
    h = jnp.maximum(p_ref[0] + p_ref[1] + b1_ref[...], 0.0)
    z = jnp.dot(h, w2_ref[...], preferred_element_type=_F32) + b2_ref[...]
    e = jnp.exp(-jnp.abs(z))
    o_ref[...] = jnp.where(z >= 0.0, 1.0 / (1.0 + e), e / (1.0 + e))


def _fc_head(xf, fw1, fb1, fw2, fb2):
    bsz, kdim = xf.shape
    hid = fw1.shape[1]
    nk_half = 14
    tk = kdim // (2 * nk_half)
    partial = pl.pallas_call(
        _fc1_kernel,
        out_shape=jax.ShapeDtypeStruct((2, bsz, hid), _F32),
        grid=(2, nk_half),
        in_specs=[
            pl.BlockSpec((bsz, tk), lambda c, k: (0, c * nk_half + k)),
            pl.BlockSpec((tk, hid), lambda c, k: (c * nk_half + k, 0)),
        ],
        out_specs=pl.BlockSpec((1, bsz, hid), lambda c, k: (c, 0, 0)),
        scratch_shapes=[pltpu.VMEM((bsz, hid), _F32)],
        compiler_params=pltpu.CompilerParams(
            dimension_semantics=("parallel", "arbitrary")),
    )(xf, fw1)
    return pl.pallas_call(
        _fc2_kernel,
        out_shape=jax.ShapeDtypeStruct((bsz, 1), _F32),
        in_specs=[
            pl.BlockSpec((2, bsz, hid), lambda: (0, 0, 0)),
            pl.BlockSpec((1, hid), lambda: (0, 0)),
            pl.BlockSpec((hid, 1), lambda: (0, 0)),
            pl.BlockSpec((1, 1), lambda: (0, 0)),
        ],
        out_specs=pl.BlockSpec((bsz, 1), lambda: (0, 0)),
    )(partial, fb1.reshape(1, hid), fw2, fb2.reshape(1, 1))


def _pool_selector(lanes, wvalid):
    r = jnp.arange(lanes)[:, None]
    c = jnp.arange(128)[None, :]
    return ((r == 2 * c) & (c < wvalid)).astype(_BF)


def _conv_weight_mat(w, cpad):
    cin = w.shape[2]
    if cpad != cin:
        w = jnp.pad(w, ((0, 0), (0, 0), (0, cpad - cin), (0, 0)))
    cout = w.shape[3]
    return w.transpose(3, 0, 1, 2).reshape(cout, 9 * cpad).astype(_BF)


def kernel(x, cw1, cb1, cw2, cb2, cw3, cb3, fw1, fb1, fw2, fb2):
    bsz = x.shape[0]
    xh = jnp.transpose(x, (0, 2, 1, 3)).astype(_BF)
    xp = jnp.pad(xh, ((0, 0), (1, 1), (0, 0), (0, 32)))

    w1m = _conv_weight_mat(cw1, 8)
    w2m = _conv_weight_mat(cw2, 32)
    w3m = _conv_weight_mat(cw3, 64)
    ident = jnp.eye(128, dtype=_BF)
    s1 = _pool_selector(256, 112)
    s2 = _pool_selector(128, 56)
    s3 = _pool_selector(128, 28)

    y1 = _conv_stage(xp, w1m, cb1.reshape(-1, 1), s1, cin=3, cpad=8,
                     hout=112, wvalid=112, pp=8)
    y2 = _conv_stage(y1, w2m, cb2.reshape(-1, 1), s2, cin=32, cpad=32,
                     hout=56, wvalid=56, pp=8)
    y3 = _conv_stage(y2, w3m, cb3.reshape(-1, 1), s3, cin=64, cpad=64,
                     hout=28, wvalid=28, pp=4, nhwc=True, ident=ident)
    xf = y3.reshape(bsz, 28 * 28 * 128)
    return _fc_head(xf, fw1, fb1, fw2, fb2)


# speedup vs baseline: 6.3664x; 1.0178x over previous
import functools

import jax
import jax.numpy as jnp
from jax.experimental import pallas as pl
from jax.experimental.pallas import tpu as pltpu

_BF = jnp.bfloat16
_F32 = jnp.float32


def _shift_lanes(v, d):
    if d == 0:
        return v
    if d < 0:
        return jnp.concatenate([v[..., 1:], v[..., :1]], axis=-1)
    return jnp.concatenate([v[..., -1:], v[..., :-1]], axis=-1)


def _build_shifted(x, cin, cpad, r_ref):
    hin2, _, lanes = x.shape
    for dx in range(3):
        s = _shift_lanes(x, 1 - dx)
        if cpad != cin:
            s = jnp.concatenate(
                [s, jnp.zeros((hin2, cpad - cin, lanes), x.dtype)], axis=1)
        r_ref[dx] = s


def _chunk_patch(r_ref, rows):
    cols = [
        jnp.concatenate(
            [r_ref[dx, y + dy] for dy in range(3) for dx in range(3)], axis=0)
        for y in rows
    ]
    return cols[0] if len(cols) == 1 else jnp.concatenate(cols, axis=1)


def _pooled_chunk(r_ref, wm, bias, sel, p0, pp, lanes):
    rows_e = [2 * (p0 + i) for i in range(pp)]
    ae = jnp.dot(wm, _chunk_patch(r_ref, rows_e), preferred_element_type=_F32)
    ao = jnp.dot(wm, _chunk_patch(r_ref, [r + 1 for r in rows_e]),
                 preferred_element_type=_F32)
    m = jnp.maximum(ae, ao)
    mm = jnp.maximum(m, _shift_lanes(m, -1))
    pooled = jnp.maximum(mm + bias, 0.0).astype(_BF)
    return [
        jnp.dot(pooled[:, i * lanes:(i + 1) * lanes], sel,
                preferred_element_type=_F32).astype(_BF)
        for i in range(pp)
    ]


def _conv_layer(r_ref, wm, bias, sel, hout, pp, lanes, store):
    for p0 in range(0, hout, pp):
        rows = _pooled_chunk(r_ref, wm, bias, sel, p0, pp, lanes)
        for i in range(pp):
            store(p0 + i, rows[i])


def _cnn_kernel(x_ref, w1_ref, b1_ref, s1_ref, w2_ref, b2_ref, s2_ref,
                w3_ref, b3_ref, s3_ref, i_ref, o_ref,
                r1_ref, y1_ref, r2_ref, y2_ref, r3_ref):
    _build_shifted(x_ref[0], 3, 8, r1_ref)
    y1_ref[0] = jnp.zeros((32, 128), _BF)
    y1_ref[113] = jnp.zeros((32, 128), _BF)
    _conv_layer(r1_ref, w1_ref[...], b1_ref[...], s1_ref[...], 112, 8, 256,
                lambda r, v: y1_ref.__setitem__(1 + r, v))

    _build_shifted(y1_ref[...], 32, 32, r2_ref)
    y2_ref[0] = jnp.zeros((64, 128), _BF)
    y2_ref[57] = jnp.zeros((64, 128), _BF)
    _conv_layer(r2_ref, w2_ref[...], b2_ref[...], s2_ref[...], 56, 8, 128,
                lambda r, v: y2_ref.__setitem__(1 + r, v))

    _build_shifted(y2_ref[...], 64, 64, r3_ref)
    w3 = w3_ref[...]
    b3 = b3_ref[...]
    s3 = s3_ref[...]
    ident = i_ref[...]
    for p0 in range(0, 28, 4):
        rows = _pooled_chunk(r3_ref, w3, b3, s3, p0, 4, 128)
        d = jnp.concatenate(rows, axis=1)
        t = jax.lax.dot_general(d, ident, (((0,), (0,)), ((), ())),
                                preferred_element_type=_F32)
        tb = t.astype(o_ref.dtype)
        for i in range(4):
            o_ref[0, p0 + i] = tb[i * 128:i * 128 + 28, :]


def _cnn_call(xp, w1m, cb1, s1, w2m, cb2, s2, w3m, cb3, s3, ident):
    bsz = xp.shape[0]
    const = lambda spec_shape: pl.BlockSpec(spec_shape, lambda b: tuple(
        0 for _ in spec_shape))
    return pl.pallas_call(
        _cnn_kernel,
        out_shape=jax.ShapeDtypeStruct((bsz, 28, 28, 128), _BF),
        grid=(bsz,),
        in_specs=[
            pl.BlockSpec((1, 226, 3, 256), lambda b: (b, 0, 0, 0)),
            const(w1m.shape), const((32, 1)), const(s1.shape),
            const(w2m.shape), const((64, 1)), const(s2.shape),
            const(w3m.shape), const((128, 1)), const(s3.shape),
            const((128, 128)),
        ],
        out_specs=pl.BlockSpec((1, 28, 28, 128), lambda b: (b, 0, 0, 0)),
        scratch_shapes=[
            pltpu.VMEM((3, 226, 8, 256), _BF),
            pltpu.VMEM((114, 32, 128), _BF),
            pltpu.VMEM((3, 114, 32, 128), _BF),
            pltpu.VMEM((58, 64, 128), _BF),
            pltpu.VMEM((3, 58, 64, 128), _BF),
        ],
        compiler_params=pltpu.CompilerParams(
            dimension_semantics=("parallel",),
            vmem_limit_bytes=100 * 1024 * 1024),
    )(xp, w1m, cb1, s1, w2m, cb2, s2, w3m, cb3, s3, ident)


def _fc1_kernel(x_ref, w_ref, o_ref, acc_ref):
    k = pl.program_id(1)

    @pl.when(k == 0)
    def _():
        acc_ref[...] = jnp.zeros(acc_ref.shape, acc_ref.dtype)

    acc_ref[...] += jnp.dot(x_ref[...], w_ref[...],
                            preferred_element_type=_F32)

    @pl.when(k == pl.num_programs(1) - 1)
    def _():
        o_ref[0] = acc_ref[...]


def _fc2_kernel(p_ref, b1_ref, w2_ref, b2_ref, o_ref):
    h = jnp.maximum(p_ref[0] + p_ref[1] + b1_ref[...], 0.0)
    z = jnp.dot(h, w2_ref[...], preferred_element_type=_F32) + b2_ref[...]
    e = jnp.exp(-jnp.abs(z))
    o_ref[...] = jnp.where(z >= 0.0, 1.0 / (1.0 + e), e / (1.0 + e))


def _fc_head(xf, fw1, fb1, fw2, fb2):
    bsz, kdim = xf.shape
    hid = fw1.shape[1]
    nk_half = 14
    tk = kdim // (2 * nk_half)
    partial = pl.pallas_call(
        _fc1_kernel,
        out_shape=jax.ShapeDtypeStruct((2, bsz, hid), _F32),
        grid=(2, nk_half),
        in_specs=[
            pl.BlockSpec((bsz, tk), lambda c, k: (0, c * nk_half + k)),
            pl.BlockSpec((tk, hid), lambda c, k: (c * nk_half + k, 0)),
        ],
        out_specs=pl.BlockSpec((1, bsz, hid), lambda c, k: (c, 0, 0)),
        scratch_shapes=[pltpu.VMEM((bsz, hid), _F32)],
        compiler_params=pltpu.CompilerParams(
            dimension_semantics=("parallel", "arbitrary")),
    )(xf, fw1)
    return pl.pallas_call(
        _fc2_kernel,
        out_shape=jax.ShapeDtypeStruct((bsz, 1), _F32),
        in_specs=[
            pl.BlockSpec((2, bsz, hid), lambda: (0, 0, 0)),
            pl.BlockSpec((1, hid), lambda: (0, 0)),
            pl.BlockSpec((hid, 1), lambda: (0, 0)),
            pl.BlockSpec((1, 1), lambda: (0, 0)),
        ],
        out_specs=pl.BlockSpec((bsz, 1), lambda: (0, 0)),
    )(partial, fb1.reshape(1, hid), fw2, fb2.reshape(1, 1))


def _pool_selector(lanes, wvalid):
    r = jnp.arange(lanes)[:, None]
    c = jnp.arange(128)[None, :]
    return ((r == 2 * c) & (c < wvalid)).astype(_BF)


def _conv_weight_mat(w, cpad):
    cin = w.shape[2]
    if cpad != cin:
        w = jnp.pad(w, ((0, 0), (0, 0), (0, cpad - cin), (0, 0)))
    cout = w.shape[3]
    return w.transpose(3, 0, 1, 2).reshape(cout, 9 * cpad).astype(_BF)


def kernel(x, cw1, cb1, cw2, cb2, cw3, cb3, fw1, fb1, fw2, fb2):
    bsz = x.shape[0]
    xh = jnp.transpose(x, (0, 2, 1, 3)).astype(_BF)
    xp = jnp.pad(xh, ((0, 0), (1, 1), (0, 0), (0, 32)))

    w1m = _conv_weight_mat(cw1, 8)
    w2m = _conv_weight_mat(cw2, 32)
    w3m = _conv_weight_mat(cw3, 64)
    ident = jnp.eye(128, dtype=_BF)
    s1 = _pool_selector(256, 112)
    s2 = _pool_selector(128, 56)
    s3 = _pool_selector(128, 28)

    y3 = _cnn_call(xp, w1m, cb1.reshape(-1, 1), s1, w2m, cb2.reshape(-1, 1),
                   s2, w3m, cb3.reshape(-1, 1), s3, ident)
    xf = y3.reshape(bsz, 28 * 28 * 128)
    return _fc_head(xf, fw1, fb1, fw2, fb2)


# speedup vs baseline: 6.3916x; 1.0040x over previous
import functools

import jax
import jax.numpy as jnp
from jax.experimental import pallas as pl
from jax.experimental.pallas import tpu as pltpu

_BF = jnp.bfloat16
_F32 = jnp.float32


def _shift_lanes(v, d):
    if d == 0:
        return v
    if d < 0:
        return jnp.concatenate([v[..., 1:], v[..., :1]], axis=-1)
    return jnp.concatenate([v[..., -1:], v[..., :-1]], axis=-1)


def _build_shifted(x, cin, cpad, r_ref):
    hin2, _, lanes = x.shape
    for dx in range(3):
        s = _shift_lanes(x, 1 - dx)
        if cpad != cin:
            s = jnp.concatenate(
                [s, jnp.zeros((hin2, cpad - cin, lanes), x.dtype)], axis=1)
        r_ref[dx] = s


def _chunk_patch(r_ref, rows):
    cols = [
        jnp.concatenate(
            [r_ref[dx, y + dy] for dy in range(3) for dx in range(3)], axis=0)
        for y in rows
    ]
    return cols[0] if len(cols) == 1 else jnp.concatenate(cols, axis=1)


def _pooled_chunk(r_ref, wm, bias, sel, p0, pp, lanes):
    rows_e = [2 * (p0 + i) for i in range(pp)]
    ae = jnp.dot(wm, _chunk_patch(r_ref, rows_e), preferred_element_type=_F32)
    ao = jnp.dot(wm, _chunk_patch(r_ref, [r + 1 for r in rows_e]),
                 preferred_element_type=_F32)
    m = jnp.maximum(ae, ao)
    mm = jnp.maximum(m, _shift_lanes(m, -1))
    pooled = jnp.maximum(mm + bias, 0.0).astype(_BF)
    return [
        jnp.dot(pooled[:, i * lanes:(i + 1) * lanes], sel,
                preferred_element_type=_F32).astype(_BF)
        for i in range(pp)
    ]


def _conv_layer(r_ref, wm, bias, sel, hout, pp, lanes, store):
    for p0 in range(0, hout, pp):
        rows = _pooled_chunk(r_ref, wm, bias, sel, p0, pp, lanes)
        for i in range(pp):
            store(p0 + i, rows[i])


def _cnn_kernel(x_ref, w1_ref, b1_ref, s1_ref, w2_ref, b2_ref, s2_ref,
                w3_ref, b3_ref, s3_ref, i_ref, o_ref,
                r1_ref, y1_ref, r2_ref, y2_ref, r3_ref):
    _build_shifted(x_ref[0], 3, 8, r1_ref)
    y1_ref[0] = jnp.zeros((32, 128), _BF)
    y1_ref[113] = jnp.zeros((32, 128), _BF)
    _conv_layer(r1_ref, w1_ref[...], b1_ref[...], s1_ref[...], 112, 8, 256,
                lambda r, v: y1_ref.__setitem__(1 + r, v))

    _build_shifted(y1_ref[...], 32, 32, r2_ref)
    y2_ref[0] = jnp.zeros((64, 128), _BF)
    y2_ref[57] = jnp.zeros((64, 128), _BF)
    _conv_layer(r2_ref, w2_ref[...], b2_ref[...], s2_ref[...], 56, 8, 128,
                lambda r, v: y2_ref.__setitem__(1 + r, v))

    _build_shifted(y2_ref[...], 64, 64, r3_ref)
    w3 = w3_ref[...]
    b3 = b3_ref[...]
    s3 = s3_ref[...]
    ident = i_ref[...]
    for p0 in range(0, 28, 4):
        rows = _pooled_chunk(r3_ref, w3, b3, s3, p0, 4, 128)
        d = jnp.concatenate(rows, axis=1)
        t = jax.lax.dot_general(d, ident, (((0,), (0,)), ((), ())),
                                preferred_element_type=_F32)
        tb = t.astype(o_ref.dtype)
        for i in range(4):
            o_ref[0, p0 + i] = tb[i * 128:i * 128 + 28, :]


def _cnn_call(xp, w1m, cb1, s1, w2m, cb2, s2, w3m, cb3, s3, ident):
    bsz = xp.shape[0]
    half = bsz // 2
    const = lambda spec_shape: pl.BlockSpec(spec_shape, lambda c, k: tuple(
        0 for _ in spec_shape))
    return pl.pallas_call(
        _cnn_kernel,
        out_shape=jax.ShapeDtypeStruct((bsz, 28, 28, 128), _BF),
        grid=(2, half),
        in_specs=[
            pl.BlockSpec((1, 226, 3, 256),
                         lambda c, k: (c * half + k, 0, 0, 0)),
            const(w1m.shape), const((32, 1)), const(s1.shape),
            const(w2m.shape), const((64, 1)), const(s2.shape),
            const(w3m.shape), const((128, 1)), const(s3.shape),
            const((128, 128)),
        ],
        out_specs=pl.BlockSpec((1, 28, 28, 128),
                               lambda c, k: (c * half + k, 0, 0, 0)),
        scratch_shapes=[
            pltpu.VMEM((3, 226, 8, 256), _BF),
            pltpu.VMEM((114, 32, 128), _BF),
            pltpu.VMEM((3, 114, 32, 128), _BF),
            pltpu.VMEM((58, 64, 128), _BF),
            pltpu.VMEM((3, 58, 64, 128), _BF),
        ],
        compiler_params=pltpu.CompilerParams(
            dimension_semantics=("parallel", "parallel"),
            vmem_limit_bytes=100 * 1024 * 1024),
    )(xp, w1m, cb1, s1, w2m, cb2, s2, w3m, cb3, s3, ident)


def _fc1_kernel(x_ref, w_ref, o_ref, acc_ref):
    k = pl.program_id(1)

    @pl.when(k == 0)
    def _():
        acc_ref[...] = jnp.zeros(acc_ref.shape, acc_ref.dtype)

    acc_ref[...] += jnp.dot(x_ref[...], w_ref[...],
                            preferred_element_type=_F32)

    @pl.when(k == pl.num_programs(1) - 1)
    def _():
        o_ref[0] = acc_ref[...]


def _fc2_kernel(p_ref, b1_ref, w2_ref, b2_ref, o_ref):
    h = jnp.maximum(p_ref[0] + p_ref[1] + b1_ref[...], 0.0)
    z = jnp.dot(h, w2_ref[...], preferred_element_type=_F32) + b2_ref[...]
    e = jnp.exp(-jnp.abs(z))
    o_ref[...] = jnp.where(z >= 0.0, 1.0 / (1.0 + e), e / (1.0 + e))


def _fc_head(xf, fw1, fb1, fw2, fb2):
    bsz, kdim = xf.shape
    hid = fw1.shape[1]
    nk_half = 14
    tk = kdim // (2 * nk_half)
    partial = pl.pallas_call(
        _fc1_kernel,
        out_shape=jax.ShapeDtypeStruct((2, bsz, hid), _F32),
        grid=(2, nk_half),
        in_specs=[
            pl.BlockSpec((bsz, tk), lambda c, k: (0, c * nk_half + k)),
            pl.BlockSpec((tk, hid), lambda c, k: (c * nk_half + k, 0)),
        ],
        out_specs=pl.BlockSpec((1, bsz, hid), lambda c, k: (c, 0, 0)),
        scratch_shapes=[pltpu.VMEM((bsz, hid), _F32)],
        compiler_params=pltpu.CompilerParams(
            dimension_semantics=("parallel", "arbitrary")),
    )(xf, fw1)
    return pl.pallas_call(
        _fc2_kernel,
        out_shape=jax.ShapeDtypeStruct((bsz, 1), _F32),
        in_specs=[
            pl.BlockSpec((2, bsz, hid), lambda: (0, 0, 0)),
            pl.BlockSpec((1, hid), lambda: (0, 0)),
            pl.BlockSpec((hid, 1), lambda: (0, 0)),
            pl.BlockSpec((1, 1), lambda: (0, 0)),
        ],
        out_specs=pl.BlockSpec((bsz, 1), lambda: (0, 0)),
    )(partial, fb1.reshape(1, hid), fw2, fb2.reshape(1, 1))


def _pool_selector(lanes, wvalid):
    r = jnp.arange(lanes)[:, None]
    c = jnp.arange(128)[None, :]
    return ((r == 2 * c) & (c < wvalid)).astype(_BF)


def _conv_weight_mat(w, cpad):
    cin = w.shape[2]
    if cpad != cin:
        w = jnp.pad(w, ((0, 0), (0, 0), (0, cpad - cin), (0, 0)))
    cout = w.shape[3]
    return w.transpose(3, 0, 1, 2).reshape(cout, 9 * cpad).astype(_BF)


def kernel(x, cw1, cb1, cw2, cb2, cw3, cb3, fw1, fb1, fw2, fb2):
    bsz = x.shape[0]
    xh = jnp.transpose(x, (0, 2, 1, 3)).astype(_BF)
    xp = jnp.pad(xh, ((0, 0), (1, 1), (0, 0), (0, 32)))

    w1m = _conv_weight_mat(cw1, 8)
    w2m = _conv_weight_mat(cw2, 32)
    w3m = _conv_weight_mat(cw3, 64)
    ident = jnp.eye(128, dtype=_BF)
    s1 = _pool_selector(256, 112)
    s2 = _pool_selector(128, 56)
    s3 = _pool_selector(128, 28)

    y3 = _cnn_call(xp, w1m, cb1.reshape(-1, 1), s1, w2m, cb2.reshape(-1, 1),
                   s2, w3m, cb3.reshape(-1, 1), s3, ident)
    xf = y3.reshape(bsz, 28 * 28 * 128)
    return _fc_head(xf, fw1, fb1, fw2, fb2)
